# baseline (speedup 1.0000x reference)
import functools

import jax
import jax.numpy as jnp
from jax import lax
from jax.experimental import pallas as pl
from jax.experimental.pallas import tpu as pltpu
from jax.experimental.pallas import tpu_sc as plsc

N = 10000
NP = 10240
E = 320000
D = 128
H = 64
R = 8

NC = 2
NS = 16
NW = NC * NS
CH = 125
NROWS = E // CH
NCH = NROWS // NW
RPT = NP // NS

_f32 = jnp.float32

_mesh = plsc.VectorSubcoreMesh(core_axis_name="c", subcore_axis_name="s")


@functools.partial(
    pl.kernel,
    out_type=jax.ShapeDtypeStruct((NC, NP, D), _f32),
    mesh=_mesh,
    scratch_types=[
        pltpu.VMEM((NCH, CH), jnp.int32),
        pltpu.VMEM((128,), _f32),
        pltpu.VMEM((RPT,), _f32),
        pltpu.VMEM((RPT, D), _f32),
        pltpu.VMEM_SHARED((NP,), _f32),
    ],
)
def _sc_deg(dst_hbm, out_hbm, idx_v, ones_v, degloc, bbuf, deg_sh):
    cid = lax.axis_index("c")
    sid = lax.axis_index("s")
    wid = sid * NC + cid

    @pl.loop(0, RPT // 16)
    def _zero(i):
        degloc[pl.ds(i * 16, 16)] = jnp.zeros((16,), _f32)

    pltpu.sync_copy(degloc, deg_sh.at[pl.ds(sid * RPT, RPT)])

    @pl.loop(0, 8)
    def _ones(i):
        ones_v[pl.ds(i * 16, 16)] = jnp.ones((16,), _f32)

    pltpu.sync_copy(dst_hbm.at[pl.ds(wid * NCH, NCH)], idx_v)
    plsc.subcore_barrier()

    @pl.loop(0, NCH)
    def _acc(j):
        pltpu.sync_copy(ones_v.at[pl.ds(0, CH)], deg_sh.at[idx_v.at[j]], add=True)

    plsc.subcore_barrier()
    pltpu.sync_copy(deg_sh.at[pl.ds(sid * RPT, RPT)], degloc)

    @pl.loop(0, RPT // 16)
    def _bcast(i):
        vals = degloc[pl.ds(i * 16, 16)]
        for l in range(16):
            row = jnp.full((16,), vals[l], _f32)
            for k in range(8):
                bbuf[i * 16 + l, pl.ds(k * 16, 16)] = row

    pltpu.sync_copy(bbuf, out_hbm.at[cid, pl.ds(sid * RPT, RPT)])


@functools.partial(
    pl.kernel,
    out_type=jax.ShapeDtypeStruct((NC, NP, D), _f32),
    mesh=_mesh,
    scratch_types=[
        pltpu.VMEM((NCH // 2, CH), jnp.int32),
        pltpu.VMEM((NCH // 2, CH), jnp.int32),
        pltpu.VMEM((CH, D), _f32),
        pltpu.VMEM((CH, D), _f32),
        pltpu.VMEM((16, D), _f32),
        pltpu.VMEM_SHARED((NP, D), _f32),
        pltpu.SemaphoreType.DMA,
        pltpu.SemaphoreType.DMA,
    ],
)
def _sc_scatter(hws_hbm, src_hbm, dst_hbm, out_hbm, sidx, didx, rows0, rows1,
                zbuf, agg_sh, sem0, sem1):
    cid = lax.axis_index("c")
    sid = lax.axis_index("s")
    wid = sid * NC + cid
    PCH = NCH // 2

    @pl.loop(0, 16)
    def _zero(i):
        for k in range(D // 16):
            zbuf[i, pl.ds(k * 16, 16)] = jnp.zeros((16,), _f32)

    @pl.loop(0, RPT // 16)
    def _zs(m):
        pltpu.sync_copy(zbuf, agg_sh.at[pl.ds(sid * RPT + m * 16, 16)])

    plsc.subcore_barrier()

    def _fire(j, buf, sem):
        pltpu.async_copy(hws_hbm.at[sidx.at[j]], buf, sem)

    def _wait(buf, sem):
        pltpu.make_async_copy(hws_hbm.at[sidx.at[0]], buf, sem).wait()

    for p in range(2):
        pltpu.sync_copy(src_hbm.at[pl.ds(wid * NCH + p * PCH, PCH)], sidx)
        pltpu.sync_copy(dst_hbm.at[pl.ds(wid * NCH + p * PCH, PCH)], didx)
        _fire(0, rows0, sem0)

        @pl.loop(0, PCH, step=2)
        def _go(j):
            _fire(j + 1, rows1, sem1)
            _wait(rows0, sem0)
            pltpu.sync_copy(rows0, agg_sh.at[didx.at[j]], add=True)

            @pl.when(j + 2 < PCH)
            def _():
                _fire(j + 2, rows0, sem0)

            _wait(rows1, sem1)
            pltpu.sync_copy(rows1, agg_sh.at[didx.at[j + 1]], add=True)

    plsc.subcore_barrier()
    pltpu.sync_copy(agg_sh.at[pl.ds(sid * RPT, RPT)],
                    out_hbm.at[cid, pl.ds(sid * RPT, RPT)])


def _make_sc_edge(nrows):
    nch = nrows // NW

    @functools.partial(
        pl.kernel,
        out_type=jax.ShapeDtypeStruct((nrows, CH, D), _f32),
        mesh=_mesh,
        scratch_types=[
            pltpu.VMEM((nch, CH), jnp.int32),
            pltpu.VMEM((nch, CH), jnp.int32),
            pltpu.VMEM((CH, D), _f32),
            pltpu.VMEM((CH, D), _f32),
            pltpu.VMEM((CH, D), _f32),
            pltpu.VMEM((CH, D), _f32),
            pltpu.VMEM((CH, D), _f32),
            pltpu.VMEM((CH, D), _f32),
            pltpu.SemaphoreType.DMA,
            pltpu.SemaphoreType.DMA,
            pltpu.SemaphoreType.DMA,
            pltpu.SemaphoreType.DMA,
        ],
    )
    def _sc_edge(h_hbm, row_hbm, col_hbm, out_hbm, ridx, kidx, ga0, gb0,
                 ga1, gb1, zo0, zo1, gs0, gs1, ss0, ss1):
        cid = lax.axis_index("c")
        sid = lax.axis_index("s")
        wid = sid * NC + cid
        base = wid * nch

        pltpu.sync_copy(row_hbm.at[pl.ds(base, nch)], ridx)
        pltpu.sync_copy(col_hbm.at[pl.ds(base, nch)], kidx)

        def _fire_g(j, ba, bb, gs):
            pltpu.async_copy(h_hbm.at[ridx.at[j]], ba, gs)
            pltpu.async_copy(h_hbm.at[kidx.at[j]], bb, gs)

        def _wait_g(ba, bb, gs):
            pltpu.make_async_copy(h_hbm.at[ridx.at[0]], ba, gs).wait()
            pltpu.make_async_copy(h_hbm.at[ridx.at[0]], bb, gs).wait()

        def _mul(ba, bb, zo):
            @pl.loop(0, CH)
            def _m(i):
                for k in range(8):
                    s = pl.ds(k * 16, 16)
                    zo[i, s] = ba[i, s] * bb[i, s]

        def _wait_s(zo, ss, j):
            pltpu.make_async_copy(zo, out_hbm.at[base + j], ss).wait()

        _fire_g(0, ga0, gb0, gs0)
        _fire_g(1, ga1, gb1, gs1)

        @pl.loop(0, nch, step=2)
        def _go(j):
            _wait_g(ga0, gb0, gs0)

            @pl.when(j >= 2)
            def _():
                _wait_s(zo0, ss0, j - 2)

            _mul(ga0, gb0, zo0)
            pltpu.async_copy(zo0, out_hbm.at[base + j], ss0)

            @pl.when(j + 2 < nch)
            def _():
                _fire_g(j + 2, ga0, gb0, gs0)

            _wait_g(ga1, gb1, gs1)

            @pl.when(j >= 2)
            def _():
                _wait_s(zo1, ss1, j - 1)

            _mul(ga1, gb1, zo1)
            pltpu.async_copy(zo1, out_hbm.at[base + j + 1], ss1)

            @pl.when(j + 3 < nch)
            def _():
                _fire_g(j + 3, ga1, gb1, gs1)

        _wait_s(zo0, ss0, nch - 2)
        _wait_s(zo1, ss1, nch - 1)

    return _sc_edge


NROWS_H = NROWS // 2
E_H = E // 2
_sc_edge_half = _make_sc_edge(NROWS_H)


BLK = 2048
GRID_N = NP // BLK
EBLK = 2000
GRID_E = E // EBLK


def _prep_body(degb_ref, x_ref, w_ref, dinv_ref, hws_ref):
    deg = jnp.maximum(degb_ref[0] + degb_ref[1] + 1.0, 1.0)
    dinv = lax.rsqrt(deg)
    dinv_ref[...] = dinv
    hw = jnp.dot(x_ref[...], w_ref[...], preferred_element_type=_f32)
    hws_ref[...] = hw * dinv


_tc_prep = pl.pallas_call(
    _prep_body,
    grid=(GRID_N,),
    in_specs=[
        pl.BlockSpec((NC, BLK, D), lambda i: (0, i, 0)),
        pl.BlockSpec((BLK, D), lambda i: (i, 0)),
        pl.BlockSpec((D, D), lambda i: (0, 0)),
    ],
    out_specs=[pl.BlockSpec((BLK, D), lambda i: (i, 0))] * 2,
    out_shape=[jax.ShapeDtypeStruct((NP, D), _f32)] * 2,
)


def _mid_body(part_ref, hws_ref, dinv_ref, b_ref, w_ref, h1_ref, hws2_ref):
    agg = dinv_ref[...] * (part_ref[0] + part_ref[1] + hws_ref[...])
    h1 = jnp.maximum(agg + b_ref[...], 0.0)
    h1_ref[...] = h1
    hws2_ref[...] = jnp.dot(h1, w_ref[...], preferred_element_type=_f32) * dinv_ref[...]


_tc_mid = pl.pallas_call(
    _mid_body,
    grid=(GRID_N,),
    in_specs=[
        pl.BlockSpec((NC, BLK, D), lambda i: (0, i, 0)),
        pl.BlockSpec((BLK, D), lambda i: (i, 0)),
        pl.BlockSpec((BLK, D), lambda i: (i, 0)),
        pl.BlockSpec((1, D), lambda i: (0, 0)),
        pl.BlockSpec((D, D), lambda i: (0, 0)),
    ],
    out_specs=[pl.BlockSpec((BLK, D), lambda i: (i, 0))] * 2,
    out_shape=[jax.ShapeDtypeStruct((NP, D), _f32)] * 2,
)


def _fin_body(part_ref, hws_ref, dinv_ref, b_ref, h1_ref, h_ref):
    h2 = jnp.maximum(
        dinv_ref[...] * (part_ref[0] + part_ref[1] + hws_ref[...]) + b_ref[...],
        0.0)
    h_ref[...] = h2 + h1_ref[...]


_tc_fin = pl.pallas_call(
    _fin_body,
    grid=(GRID_N,),
    in_specs=[
        pl.BlockSpec((NC, BLK, D), lambda i: (0, i, 0)),
        pl.BlockSpec((BLK, D), lambda i: (i, 0)),
        pl.BlockSpec((BLK, D), lambda i: (i, 0)),
        pl.BlockSpec((1, D), lambda i: (0, 0)),
        pl.BlockSpec((BLK, D), lambda i: (i, 0)),
    ],
    out_specs=pl.BlockSpec((BLK, D), lambda i: (i, 0)),
    out_shape=jax.ShapeDtypeStruct((NP, D), _f32),
)


def _mlp_body(z_ref, rel_ref, ws_ref, bs_ref, wall_ref, br_ref, out_ref):
    a = jnp.maximum(
        jnp.dot(z_ref[...], ws_ref[...], preferred_element_type=_f32) + bs_ref[...],
        0.0)
    t = jnp.dot(a, wall_ref[...], preferred_element_type=_f32) + br_ref[...]
    oh = rel_ref[...] == lax.broadcasted_iota(jnp.int32, (EBLK, R), 1)
    out_ref[...] = jnp.sum(jnp.where(oh, t, 0.0), axis=1, keepdims=True)


_tc_mlp = pl.pallas_call(
    _mlp_body,
    grid=(E_H // EBLK,),
    in_specs=[
        pl.BlockSpec((EBLK, D), lambda i: (i, 0)),
        pl.BlockSpec((EBLK, 1), lambda i: (i, 0)),
        pl.BlockSpec((D, H), lambda i: (0, 0)),
        pl.BlockSpec((1, H), lambda i: (0, 0)),
        pl.BlockSpec((H, R), lambda i: (0, 0)),
        pl.BlockSpec((1, R), lambda i: (0, 0)),
    ],
    out_specs=pl.BlockSpec((EBLK, 1), lambda i: (i, 0)),
    out_shape=jax.ShapeDtypeStruct((E_H, 1), _f32),
)


def kernel(x, edge_index, relations, train_edge_index, W1, b1, W2, b2,
           Ws, bs, Wr, br):
    src2d = train_edge_index[0].astype(jnp.int32).reshape(NROWS, CH)
    dst2d = train_edge_index[1].astype(jnp.int32).reshape(NROWS, CH)
    row2d = edge_index[:, 0].astype(jnp.int32).reshape(NROWS, CH)
    col2d = edge_index[:, 1].astype(jnp.int32).reshape(NROWS, CH)
    rel2 = relations.astype(jnp.int32).reshape(E, 1)

    x_p = jnp.pad(x, ((0, NP - N), (0, 0)))
    b1r = b1.reshape(1, D)
    b2r = b2.reshape(1, D)
    bsr = bs.reshape(1, H)
    wall = Wr.reshape(R, H).T
    brr = br.reshape(1, R)

    degb = _sc_deg(dst2d)
    dinvb, hws1 = _tc_prep(degb, x_p, W1)
    part1 = _sc_scatter(hws1, src2d, dst2d)
    h1, hws2 = _tc_mid(part1, hws1, dinvb, b1r, W2)
    part2 = _sc_scatter(hws2, src2d, dst2d)
    h = _tc_fin(part2, hws2, dinvb, b2r, h1)
    outs = []
    for k in range(2):
        rs = row2d[k * NROWS_H:(k + 1) * NROWS_H]
        cs = col2d[k * NROWS_H:(k + 1) * NROWS_H]
        z3 = _sc_edge_half(h, rs, cs)
        z = z3.reshape(E_H, D)
        outs.append(_tc_mlp(z, rel2[k * E_H:(k + 1) * E_H], Ws, bsr, wall, brr))
    return jnp.concatenate(outs, axis=0)


# speedup vs baseline: 1.0161x; 1.0161x over previous
import functools

import jax
import jax.numpy as jnp
from jax import lax
from jax.experimental import pallas as pl
from jax.experimental.pallas import tpu as pltpu
from jax.experimental.pallas import tpu_sc as plsc

N = 10000
NP = 10240
E = 320000
D = 128
H = 64
R = 8

NC = 2
NS = 16
NW = NC * NS
CH = 125
NROWS = E // CH
NCH = NROWS // NW
RPT = NP // NS

_f32 = jnp.float32

_mesh = plsc.VectorSubcoreMesh(core_axis_name="c", subcore_axis_name="s")


@functools.partial(
    pl.kernel,
    out_type=jax.ShapeDtypeStruct((NC, NP, D), _f32),
    mesh=_mesh,
    scratch_types=[
        pltpu.VMEM((NCH, CH), jnp.int32),
        pltpu.VMEM((128,), _f32),
        pltpu.VMEM((RPT,), _f32),
        pltpu.VMEM((RPT, D), _f32),
        pltpu.VMEM_SHARED((NP,), _f32),
    ],
)
def _sc_deg(dst_hbm, out_hbm, idx_v, ones_v, degloc, bbuf, deg_sh):
    cid = lax.axis_index("c")
    sid = lax.axis_index("s")
    wid = sid * NC + cid

    @pl.loop(0, RPT // 16)
    def _zero(i):
        degloc[pl.ds(i * 16, 16)] = jnp.zeros((16,), _f32)

    pltpu.sync_copy(degloc, deg_sh.at[pl.ds(sid * RPT, RPT)])

    @pl.loop(0, 8)
    def _ones(i):
        ones_v[pl.ds(i * 16, 16)] = jnp.ones((16,), _f32)

    pltpu.sync_copy(dst_hbm.at[pl.ds(wid * NCH, NCH)], idx_v)
    plsc.subcore_barrier()

    @pl.loop(0, NCH)
    def _acc(j):
        pltpu.sync_copy(ones_v.at[pl.ds(0, CH)], deg_sh.at[idx_v.at[j]], add=True)

    plsc.subcore_barrier()
    pltpu.sync_copy(deg_sh.at[pl.ds(sid * RPT, RPT)], degloc)

    @pl.loop(0, RPT // 16)
    def _bcast(i):
        vals = degloc[pl.ds(i * 16, 16)]
        for l in range(16):
            row = jnp.full((16,), vals[l], _f32)
            for k in range(8):
                bbuf[i * 16 + l, pl.ds(k * 16, 16)] = row

    pltpu.sync_copy(bbuf, out_hbm.at[cid, pl.ds(sid * RPT, RPT)])


@functools.partial(
    pl.kernel,
    out_type=jax.ShapeDtypeStruct((NC, NP, D), _f32),
    mesh=_mesh,
    scratch_types=[
        pltpu.VMEM((NCH // 2, CH), jnp.int32),
        pltpu.VMEM((NCH // 2, CH), jnp.int32),
        pltpu.VMEM((CH, D), _f32),
        pltpu.VMEM((CH, D), _f32),
        pltpu.VMEM((16, D), _f32),
        pltpu.VMEM_SHARED((NP, D), _f32),
        pltpu.SemaphoreType.DMA,
        pltpu.SemaphoreType.DMA,
        pltpu.SemaphoreType.DMA,
        pltpu.SemaphoreType.DMA,
    ],
)
def _sc_scatter(hws_hbm, src_hbm, dst_hbm, out_hbm, sidx, didx, rows0, rows1,
                zbuf, agg_sh, sem0, sem1, asem0, asem1):
    cid = lax.axis_index("c")
    sid = lax.axis_index("s")
    wid = sid * NC + cid
    PCH = NCH // 2

    @pl.loop(0, 16)
    def _zero(i):
        for k in range(D // 16):
            zbuf[i, pl.ds(k * 16, 16)] = jnp.zeros((16,), _f32)

    @pl.loop(0, RPT // 16)
    def _zs(m):
        pltpu.sync_copy(zbuf, agg_sh.at[pl.ds(sid * RPT + m * 16, 16)])

    plsc.subcore_barrier()

    def _fire(j, buf, sem):
        pltpu.async_copy(hws_hbm.at[sidx.at[j]], buf, sem)

    def _wait(buf, sem):
        pltpu.make_async_copy(hws_hbm.at[sidx.at[0]], buf, sem).wait()

    def _fire_add(j, buf, asem):
        pltpu.async_copy(buf, agg_sh.at[didx.at[j]], asem, add=True)

    def _wait_add(buf, asem):
        pltpu.make_async_copy(buf, agg_sh.at[didx.at[0]], asem).wait()

    for p in range(2):
        pltpu.sync_copy(src_hbm.at[pl.ds(wid * NCH + p * PCH, PCH)], sidx)
        pltpu.sync_copy(dst_hbm.at[pl.ds(wid * NCH + p * PCH, PCH)], didx)
        _fire(0, rows0, sem0)

        @pl.loop(0, PCH, step=2)
        def _go(j):
            _fire(j + 1, rows1, sem1)
            _wait(rows0, sem0)
            _fire_add(j, rows0, asem0)

            _wait(rows1, sem1)
            _fire_add(j + 1, rows1, asem1)

            _wait_add(rows0, asem0)

            @pl.when(j + 2 < PCH)
            def _():
                _fire(j + 2, rows0, sem0)

            _wait_add(rows1, asem1)

    plsc.subcore_barrier()
    pltpu.sync_copy(agg_sh.at[pl.ds(sid * RPT, RPT)],
                    out_hbm.at[cid, pl.ds(sid * RPT, RPT)])


def _make_sc_edge(nrows):
    nch = nrows // NW

    @functools.partial(
        pl.kernel,
        out_type=jax.ShapeDtypeStruct((nrows, CH, D), _f32),
        mesh=_mesh,
        scratch_types=[
            pltpu.VMEM((nch, CH), jnp.int32),
            pltpu.VMEM((nch, CH), jnp.int32),
            pltpu.VMEM((CH, D), _f32),
            pltpu.VMEM((CH, D), _f32),
            pltpu.VMEM((CH, D), _f32),
            pltpu.VMEM((CH, D), _f32),
            pltpu.VMEM((CH, D), _f32),
            pltpu.VMEM((CH, D), _f32),
            pltpu.SemaphoreType.DMA,
            pltpu.SemaphoreType.DMA,
            pltpu.SemaphoreType.DMA,
            pltpu.SemaphoreType.DMA,
        ],
    )
    def _sc_edge(h_hbm, row_hbm, col_hbm, out_hbm, ridx, kidx, ga0, gb0,
                 ga1, gb1, zo0, zo1, gs0, gs1, ss0, ss1):
        cid = lax.axis_index("c")
        sid = lax.axis_index("s")
        wid = sid * NC + cid
        base = wid * nch

        pltpu.sync_copy(row_hbm.at[pl.ds(base, nch)], ridx)
        pltpu.sync_copy(col_hbm.at[pl.ds(base, nch)], kidx)

        def _fire_g(j, ba, bb, gs):
            pltpu.async_copy(h_hbm.at[ridx.at[j]], ba, gs)
            pltpu.async_copy(h_hbm.at[kidx.at[j]], bb, gs)

        def _wait_g(ba, bb, gs):
            pltpu.make_async_copy(h_hbm.at[ridx.at[0]], ba, gs).wait()
            pltpu.make_async_copy(h_hbm.at[ridx.at[0]], bb, gs).wait()

        def _mul(ba, bb, zo):
            @pl.loop(0, CH)
            def _m(i):
                for k in range(8):
                    s = pl.ds(k * 16, 16)
                    zo[i, s] = ba[i, s] * bb[i, s]

        def _wait_s(zo, ss, j):
            pltpu.make_async_copy(zo, out_hbm.at[base + j], ss).wait()

        _fire_g(0, ga0, gb0, gs0)
        _fire_g(1, ga1, gb1, gs1)

        @pl.loop(0, nch, step=2)
        def _go(j):
            _wait_g(ga0, gb0, gs0)

            @pl.when(j >= 2)
            def _():
                _wait_s(zo0, ss0, j - 2)

            _mul(ga0, gb0, zo0)
            pltpu.async_copy(zo0, out_hbm.at[base + j], ss0)

            @pl.when(j + 2 < nch)
            def _():
                _fire_g(j + 2, ga0, gb0, gs0)

            _wait_g(ga1, gb1, gs1)

            @pl.when(j >= 2)
            def _():
                _wait_s(zo1, ss1, j - 1)

            _mul(ga1, gb1, zo1)
            pltpu.async_copy(zo1, out_hbm.at[base + j + 1], ss1)

            @pl.when(j + 3 < nch)
            def _():
                _fire_g(j + 3, ga1, gb1, gs1)

        _wait_s(zo0, ss0, nch - 2)
        _wait_s(zo1, ss1, nch - 1)

    return _sc_edge


_sc_edge = _make_sc_edge(NROWS)


BLK = 2048
GRID_N = NP // BLK
EBLK = 2000
GRID_E = E // EBLK


def _prep_body(degb_ref, x_ref, w_ref, dinv_ref, hws_ref):
    deg = jnp.maximum(degb_ref[0] + degb_ref[1] + 1.0, 1.0)
    dinv = lax.rsqrt(deg)
    dinv_ref[...] = dinv
    hw = jnp.dot(x_ref[...], w_ref[...], preferred_element_type=_f32)
    hws_ref[...] = hw * dinv


_tc_prep = pl.pallas_call(
    _prep_body,
    grid=(GRID_N,),
    in_specs=[
        pl.BlockSpec((NC, BLK, D), lambda i: (0, i, 0)),
        pl.BlockSpec((BLK, D), lambda i: (i, 0)),
        pl.BlockSpec((D, D), lambda i: (0, 0)),
    ],
    out_specs=[pl.BlockSpec((BLK, D), lambda i: (i, 0))] * 2,
    out_shape=[jax.ShapeDtypeStruct((NP, D), _f32)] * 2,
)


def _mid_body(part_ref, hws_ref, dinv_ref, b_ref, w_ref, h1_ref, hws2_ref):
    agg = dinv_ref[...] * (part_ref[0] + part_ref[1] + hws_ref[...])
    h1 = jnp.maximum(agg + b_ref[...], 0.0)
    h1_ref[...] = h1
    hws2_ref[...] = jnp.dot(h1, w_ref[...], preferred_element_type=_f32) * dinv_ref[...]


_tc_mid = pl.pallas_call(
    _mid_body,
    grid=(GRID_N,),
    in_specs=[
        pl.BlockSpec((NC, BLK, D), lambda i: (0, i, 0)),
        pl.BlockSpec((BLK, D), lambda i: (i, 0)),
        pl.BlockSpec((BLK, D), lambda i: (i, 0)),
        pl.BlockSpec((1, D), lambda i: (0, 0)),
        pl.BlockSpec((D, D), lambda i: (0, 0)),
    ],
    out_specs=[pl.BlockSpec((BLK, D), lambda i: (i, 0))] * 2,
    out_shape=[jax.ShapeDtypeStruct((NP, D), _f32)] * 2,
)


def _fin_body(part_ref, hws_ref, dinv_ref, b_ref, h1_ref, h_ref):
    h2 = jnp.maximum(
        dinv_ref[...] * (part_ref[0] + part_ref[1] + hws_ref[...]) + b_ref[...],
        0.0)
    h_ref[...] = h2 + h1_ref[...]


_tc_fin = pl.pallas_call(
    _fin_body,
    grid=(GRID_N,),
    in_specs=[
        pl.BlockSpec((NC, BLK, D), lambda i: (0, i, 0)),
        pl.BlockSpec((BLK, D), lambda i: (i, 0)),
        pl.BlockSpec((BLK, D), lambda i: (i, 0)),
        pl.BlockSpec((1, D), lambda i: (0, 0)),
        pl.BlockSpec((BLK, D), lambda i: (i, 0)),
    ],
    out_specs=pl.BlockSpec((BLK, D), lambda i: (i, 0)),
    out_shape=jax.ShapeDtypeStruct((NP, D), _f32),
)


def _mlp_body(z_ref, rel_ref, ws_ref, bs_ref, wall_ref, br_ref, out_ref):
    a = jnp.maximum(
        jnp.dot(z_ref[...], ws_ref[...], preferred_element_type=_f32) + bs_ref[...],
        0.0)
    t = jnp.dot(a, wall_ref[...], preferred_element_type=_f32) + br_ref[...]
    oh = rel_ref[...] == lax.broadcasted_iota(jnp.int32, (EBLK, R), 1)
    out_ref[...] = jnp.sum(jnp.where(oh, t, 0.0), axis=1, keepdims=True)


_tc_mlp = pl.pallas_call(
    _mlp_body,
    grid=(GRID_E,),
    in_specs=[
        pl.BlockSpec((EBLK, D), lambda i: (i, 0)),
        pl.BlockSpec((EBLK, 1), lambda i: (i, 0)),
        pl.BlockSpec((D, H), lambda i: (0, 0)),
        pl.BlockSpec((1, H), lambda i: (0, 0)),
        pl.BlockSpec((H, R), lambda i: (0, 0)),
        pl.BlockSpec((1, R), lambda i: (0, 0)),
    ],
    out_specs=pl.BlockSpec((EBLK, 1), lambda i: (i, 0)),
    out_shape=jax.ShapeDtypeStruct((E, 1), _f32),
)


def kernel(x, edge_index, relations, train_edge_index, W1, b1, W2, b2,
           Ws, bs, Wr, br):
    src2d = train_edge_index[0].astype(jnp.int32).reshape(NROWS, CH)
    dst2d = train_edge_index[1].astype(jnp.int32).reshape(NROWS, CH)
    row2d = edge_index[:, 0].astype(jnp.int32).reshape(NROWS, CH)
    col2d = edge_index[:, 1].astype(jnp.int32).reshape(NROWS, CH)
    rel2 = relations.astype(jnp.int32).reshape(E, 1)

    x_p = jnp.pad(x, ((0, NP - N), (0, 0)))
    b1r = b1.reshape(1, D)
    b2r = b2.reshape(1, D)
    bsr = bs.reshape(1, H)
    wall = Wr.reshape(R, H).T
    brr = br.reshape(1, R)

    degb = _sc_deg(dst2d)
    dinvb, hws1 = _tc_prep(degb, x_p, W1)
    part1 = _sc_scatter(hws1, src2d, dst2d)
    h1, hws2 = _tc_mid(part1, hws1, dinvb, b1r, W2)
    part2 = _sc_scatter(hws2, src2d, dst2d)
    h = _tc_fin(part2, hws2, dinvb, b2r, h1)
    z3 = _sc_edge(h, row2d, col2d)
    z = z3.reshape(E, D)
    out = _tc_mlp(z, rel2, Ws, bsr, wall, brr)
    return out


# speedup vs baseline: 1.0911x; 1.0739x over previous
import functools

import jax
import jax.numpy as jnp
from jax import lax
from jax.experimental import pallas as pl
from jax.experimental.pallas import tpu as pltpu
from jax.experimental.pallas import tpu_sc as plsc

N = 10000
NP = 10240
E = 320000
D = 128
H = 64
R = 8

NC = 2
NS = 16
NW = NC * NS
CH = 125
NROWS = E // CH
NCH = NROWS // NW
RPT = NP // NS

_f32 = jnp.float32

_mesh = plsc.VectorSubcoreMesh(core_axis_name="c", subcore_axis_name="s")


@functools.partial(
    pl.kernel,
    out_type=jax.ShapeDtypeStruct((NC, NP, D), _f32),
    mesh=_mesh,
    scratch_types=[
        pltpu.VMEM((NCH, CH), jnp.int32),
        pltpu.VMEM((128,), _f32),
        pltpu.VMEM((RPT,), _f32),
        pltpu.VMEM((RPT, D), _f32),
        pltpu.VMEM_SHARED((NP,), _f32),
    ],
)
def _sc_deg(dst_hbm, out_hbm, idx_v, ones_v, degloc, bbuf, deg_sh):
    cid = lax.axis_index("c")
    sid = lax.axis_index("s")
    wid = sid * NC + cid

    @pl.loop(0, RPT // 16)
    def _zero(i):
        degloc[pl.ds(i * 16, 16)] = jnp.zeros((16,), _f32)

    pltpu.sync_copy(degloc, deg_sh.at[pl.ds(sid * RPT, RPT)])

    @pl.loop(0, 8)
    def _ones(i):
        ones_v[pl.ds(i * 16, 16)] = jnp.ones((16,), _f32)

    pltpu.sync_copy(dst_hbm.at[pl.ds(wid * NCH, NCH)], idx_v)
    plsc.subcore_barrier()

    @pl.loop(0, NCH)
    def _acc(j):
        pltpu.sync_copy(ones_v.at[pl.ds(0, CH)], deg_sh.at[idx_v.at[j]], add=True)

    plsc.subcore_barrier()
    pltpu.sync_copy(deg_sh.at[pl.ds(sid * RPT, RPT)], degloc)

    @pl.loop(0, RPT // 16)
    def _bcast(i):
        vals = degloc[pl.ds(i * 16, 16)]
        for l in range(16):
            row = jnp.full((16,), vals[l], _f32)
            for k in range(8):
                bbuf[i * 16 + l, pl.ds(k * 16, 16)] = row

    pltpu.sync_copy(bbuf, out_hbm.at[cid, pl.ds(sid * RPT, RPT)])


@functools.partial(
    pl.kernel,
    out_type=jax.ShapeDtypeStruct((NC, NP, D), _f32),
    mesh=_mesh,
    scratch_types=[
        pltpu.VMEM((NCH // 2, CH), jnp.int32),
        pltpu.VMEM((NCH // 2, CH), jnp.int32),
        pltpu.VMEM((CH, D), _f32),
        pltpu.VMEM((CH, D), _f32),
        pltpu.VMEM((16, D), _f32),
        pltpu.VMEM_SHARED((NP, D), _f32),
        pltpu.SemaphoreType.DMA,
        pltpu.SemaphoreType.DMA,
    ],
)
def _sc_scatter(hws_hbm, src_hbm, dst_hbm, out_hbm, sidx, didx, rows0, rows1,
                zbuf, agg_sh, sem0, sem1):
    cid = lax.axis_index("c")
    sid = lax.axis_index("s")
    wid = sid * NC + cid
    PCH = NCH // 2

    @pl.loop(0, 16)
    def _zero(i):
        for k in range(D // 16):
            zbuf[i, pl.ds(k * 16, 16)] = jnp.zeros((16,), _f32)

    @pl.loop(0, RPT // 16)
    def _zs(m):
        pltpu.sync_copy(zbuf, agg_sh.at[pl.ds(sid * RPT + m * 16, 16)])

    plsc.subcore_barrier()

    def _fire(j, buf, sem):
        pltpu.async_copy(hws_hbm.at[sidx.at[j]], buf, sem)

    def _wait(buf, sem):
        pltpu.make_async_copy(hws_hbm.at[sidx.at[0]], buf, sem).wait()

    for p in range(2):
        pltpu.sync_copy(src_hbm.at[pl.ds(wid * NCH + p * PCH, PCH)], sidx)
        pltpu.sync_copy(dst_hbm.at[pl.ds(wid * NCH + p * PCH, PCH)], didx)
        _fire(0, rows0, sem0)

        @pl.loop(0, PCH, step=2)
        def _go(j):
            _fire(j + 1, rows1, sem1)
            _wait(rows0, sem0)
            pltpu.sync_copy(rows0, agg_sh.at[didx.at[j]], add=True)

            @pl.when(j + 2 < PCH)
            def _():
                _fire(j + 2, rows0, sem0)

            _wait(rows1, sem1)
            pltpu.sync_copy(rows1, agg_sh.at[didx.at[j + 1]], add=True)

    plsc.subcore_barrier()
    pltpu.sync_copy(agg_sh.at[pl.ds(sid * RPT, RPT)],
                    out_hbm.at[cid, pl.ds(sid * RPT, RPT)])


def _make_sc_edge(nrows):
    nch = nrows // NW

    @functools.partial(
        pl.kernel,
        out_type=jax.ShapeDtypeStruct((nrows, CH, D), _f32),
        mesh=_mesh,
        scratch_types=[
            pltpu.VMEM((nch, CH), jnp.int32),
            pltpu.VMEM((nch, CH), jnp.int32),
            pltpu.VMEM((CH, D), _f32),
            pltpu.VMEM((CH, D), _f32),
            pltpu.VMEM((CH, D), _f32),
            pltpu.VMEM((CH, D), _f32),
            pltpu.VMEM((CH, D), _f32),
            pltpu.VMEM((CH, D), _f32),
            pltpu.SemaphoreType.DMA,
            pltpu.SemaphoreType.DMA,
            pltpu.SemaphoreType.DMA,
            pltpu.SemaphoreType.DMA,
        ],
    )
    def _sc_edge(h_hbm, row_hbm, col_hbm, out_hbm, ridx, kidx, ga0, gb0,
                 ga1, gb1, zo0, zo1, gs0, gs1, ss0, ss1):
        cid = lax.axis_index("c")
        sid = lax.axis_index("s")
        wid = sid * NC + cid
        base = wid * nch

        pltpu.sync_copy(row_hbm.at[pl.ds(base, nch)], ridx)
        pltpu.sync_copy(col_hbm.at[pl.ds(base, nch)], kidx)

        def _fire_g(j, ba, bb, gs):
            pltpu.async_copy(h_hbm.at[ridx.at[j]], ba, gs)
            pltpu.async_copy(h_hbm.at[kidx.at[j]], bb, gs)

        def _wait_g(ba, bb, gs):
            pltpu.make_async_copy(h_hbm.at[ridx.at[0]], ba, gs).wait()
            pltpu.make_async_copy(h_hbm.at[ridx.at[0]], bb, gs).wait()

        def _mul(ba, bb, zo):
            @pl.loop(0, CH)
            def _m(i):
                for k in range(8):
                    s = pl.ds(k * 16, 16)
                    zo[i, s] = ba[i, s] * bb[i, s]

        def _wait_s(zo, ss, j):
            pltpu.make_async_copy(zo, out_hbm.at[base + j], ss).wait()

        _fire_g(0, ga0, gb0, gs0)
        _fire_g(1, ga1, gb1, gs1)

        @pl.loop(0, nch, step=2)
        def _go(j):
            _wait_g(ga0, gb0, gs0)

            @pl.when(j >= 2)
            def _():
                _wait_s(zo0, ss0, j - 2)

            _mul(ga0, gb0, zo0)
            pltpu.async_copy(zo0, out_hbm.at[base + j], ss0)

            @pl.when(j + 2 < nch)
            def _():
                _fire_g(j + 2, ga0, gb0, gs0)

            _wait_g(ga1, gb1, gs1)

            @pl.when(j >= 2)
            def _():
                _wait_s(zo1, ss1, j - 1)

            _mul(ga1, gb1, zo1)
            pltpu.async_copy(zo1, out_hbm.at[base + j + 1], ss1)

            @pl.when(j + 3 < nch)
            def _():
                _fire_g(j + 3, ga1, gb1, gs1)

        _wait_s(zo0, ss0, nch - 2)
        _wait_s(zo1, ss1, nch - 1)

    return _sc_edge


_sc_edge = _make_sc_edge(NROWS)


BLK = 2048
GRID_N = NP // BLK
EBLK = 2000
GRID_E = E // EBLK


def _prep_body(degb_ref, x_ref, w_ref, dinv_ref, hws_ref):
    deg = jnp.maximum(degb_ref[0] + degb_ref[1] + 1.0, 1.0)
    dinv = lax.rsqrt(deg)
    dinv_ref[...] = dinv
    hw = jnp.dot(x_ref[...], w_ref[...], preferred_element_type=_f32)
    hws_ref[...] = hw * dinv


_tc_prep = pl.pallas_call(
    _prep_body,
    grid=(GRID_N,),
    in_specs=[
        pl.BlockSpec((NC, BLK, D), lambda i: (0, i, 0)),
        pl.BlockSpec((BLK, D), lambda i: (i, 0)),
        pl.BlockSpec((D, D), lambda i: (0, 0)),
    ],
    out_specs=[pl.BlockSpec((BLK, D), lambda i: (i, 0))] * 2,
    out_shape=[jax.ShapeDtypeStruct((NP, D), _f32)] * 2,
)


def _mid_body(part_ref, hws_ref, dinv_ref, b_ref, w_ref, h1_ref, hws2_ref):
    agg = dinv_ref[...] * (part_ref[0] + part_ref[1] + hws_ref[...])
    h1 = jnp.maximum(agg + b_ref[...], 0.0)
    h1_ref[...] = h1
    hws2_ref[...] = jnp.dot(h1, w_ref[...], preferred_element_type=_f32) * dinv_ref[...]


_tc_mid = pl.pallas_call(
    _mid_body,
    grid=(GRID_N,),
    in_specs=[
        pl.BlockSpec((NC, BLK, D), lambda i: (0, i, 0)),
        pl.BlockSpec((BLK, D), lambda i: (i, 0)),
        pl.BlockSpec((BLK, D), lambda i: (i, 0)),
        pl.BlockSpec((1, D), lambda i: (0, 0)),
        pl.BlockSpec((D, D), lambda i: (0, 0)),
    ],
    out_specs=[pl.BlockSpec((BLK, D), lambda i: (i, 0))] * 2,
    out_shape=[jax.ShapeDtypeStruct((NP, D), _f32)] * 2,
)


def _fin_body(part_ref, hws_ref, dinv_ref, b_ref, h1_ref, h_ref):
    h2 = jnp.maximum(
        dinv_ref[...] * (part_ref[0] + part_ref[1] + hws_ref[...]) + b_ref[...],
        0.0)
    h_ref[...] = h2 + h1_ref[...]


_tc_fin = pl.pallas_call(
    _fin_body,
    grid=(GRID_N,),
    in_specs=[
        pl.BlockSpec((NC, BLK, D), lambda i: (0, i, 0)),
        pl.BlockSpec((BLK, D), lambda i: (i, 0)),
        pl.BlockSpec((BLK, D), lambda i: (i, 0)),
        pl.BlockSpec((1, D), lambda i: (0, 0)),
        pl.BlockSpec((BLK, D), lambda i: (i, 0)),
    ],
    out_specs=pl.BlockSpec((BLK, D), lambda i: (i, 0)),
    out_shape=jax.ShapeDtypeStruct((NP, D), _f32),
)


def _mlp_body(z_ref, rel_ref, ws_ref, bs_ref, wall_ref, br_ref, out_ref):
    a = jnp.maximum(
        jnp.dot(z_ref[...], ws_ref[...], preferred_element_type=_f32) + bs_ref[...],
        0.0)
    t = jnp.dot(a, wall_ref[...], preferred_element_type=_f32) + br_ref[...]
    oh = rel_ref[...] == lax.broadcasted_iota(jnp.int32, (EBLK, R), 1)
    out_ref[...] = jnp.sum(jnp.where(oh, t, 0.0), axis=1, keepdims=True)


_tc_mlp = pl.pallas_call(
    _mlp_body,
    grid=(GRID_E,),
    in_specs=[
        pl.BlockSpec((EBLK, D), lambda i: (i, 0)),
        pl.BlockSpec((EBLK, 1), lambda i: (i, 0)),
        pl.BlockSpec((D, H), lambda i: (0, 0)),
        pl.BlockSpec((1, H), lambda i: (0, 0)),
        pl.BlockSpec((H, R), lambda i: (0, 0)),
        pl.BlockSpec((1, R), lambda i: (0, 0)),
    ],
    out_specs=pl.BlockSpec((EBLK, 1), lambda i: (i, 0)),
    out_shape=jax.ShapeDtypeStruct((E, 1), _f32),
)


def kernel(x, edge_index, relations, train_edge_index, W1, b1, W2, b2,
           Ws, bs, Wr, br):
    src2d = train_edge_index[0].astype(jnp.int32).reshape(NROWS, CH)
    dst2d = train_edge_index[1].astype(jnp.int32).reshape(NROWS, CH)
    row2d = edge_index[:, 0].astype(jnp.int32).reshape(NROWS, CH)
    col2d = edge_index[:, 1].astype(jnp.int32).reshape(NROWS, CH)
    rel2 = relations.astype(jnp.int32).reshape(E, 1)

    x_p = jnp.pad(x, ((0, NP - N), (0, 0)))
    b1r = b1.reshape(1, D)
    b2r = b2.reshape(1, D)
    bsr = bs.reshape(1, H)
    wall = Wr.reshape(R, H).T
    brr = br.reshape(1, R)

    degb = _sc_deg(dst2d)
    dinvb, hws1 = _tc_prep(degb, x_p, W1)
    part1 = _sc_scatter(hws1, src2d, dst2d)
    h1, hws2 = _tc_mid(part1, hws1, dinvb, b1r, W2)
    part2 = _sc_scatter(hws2, src2d, dst2d)
    h = _tc_fin(part2, hws2, dinvb, b2r, h1)
    z3 = _sc_edge(h, row2d, col2d)
    z = z3.reshape(E, D)
    out = _tc_mlp(z, rel2, Ws, bsr, wall, brr)
    return out
